# initial kernel scaffold (unmeasured)
import jax
import jax.numpy as jnp
from jax import lax
from jax.experimental import pallas as pl
from jax.experimental.pallas import tpu as pltpu


def _comm_body(k_ref, v_ref, kf_ref, vf_ref, send_sems, recv_sems):
    my_x = lax.axis_index("x")
    my_y = lax.axis_index("y")
    my_z = lax.axis_index("z")
    other = (1 - my_x, my_y, my_z)

    s_half = k_ref.shape[0]
    offs = my_x * s_half

    kf_ref[pl.ds(offs, s_half)] = k_ref[...].astype(jnp.bfloat16)
    vf_ref[pl.ds(offs, s_half)] = v_ref[...].astype(jnp.bfloat16)

    barrier_sem = pltpu.get_barrier_semaphore()
    pl.semaphore_signal(
        barrier_sem, inc=1, device_id=other,
        device_id_type=pl.DeviceIdType.MESH,
    )
    pl.semaphore_wait(barrier_sem, 1)

    rdma_k = pltpu.make_async_remote_copy(
        src_ref=kf_ref.at[pl.ds(offs, s_half)],
        dst_ref=kf_ref.at[pl.ds(offs, s_half)],
        send_sem=send_sems.at[0],
        recv_sem=recv_sems.at[0],
        device_id=other,
        device_id_type=pl.DeviceIdType.MESH,
    )
    rdma_v = pltpu.make_async_remote_copy(
        src_ref=vf_ref.at[pl.ds(offs, s_half)],
        dst_ref=vf_ref.at[pl.ds(offs, s_half)],
        send_sem=send_sems.at[1],
        recv_sem=recv_sems.at[1],
        device_id=other,
        device_id_type=pl.DeviceIdType.MESH,
    )
    rdma_k.start()
    rdma_v.start()
    rdma_k.wait()
    rdma_v.wait()


def _attn_body(q_ref, kf_ref, vf_ref, o_ref):
    d = q_ref.shape[-1]
    q = (q_ref[:, 0, :] * (d ** -0.5)).astype(jnp.bfloat16)
    k = kf_ref[:, 0, :]
    s = lax.dot_general(
        q, k, (((1,), (1,)), ((), ())), preferred_element_type=jnp.float32
    )
    m = jnp.max(s, axis=1, keepdims=True)
    p = jnp.exp(s - m)
    l = jnp.sum(p, axis=1, keepdims=True)
    v = vf_ref[:, 0, :]
    o = lax.dot_general(
        p.astype(jnp.bfloat16), v, (((1,), (0,)), ((), ())),
        preferred_element_type=jnp.float32,
    )
    o_ref[:, 0, :] = o / l


def kernel(Q, K, V):
    q = Q[0]
    k = K[0]
    v = V[0]
    s_half, h, d = q.shape
    s_full = 2 * s_half

    kf, vf = pl.pallas_call(
        _comm_body,
        out_shape=(
            jax.ShapeDtypeStruct((s_full, h, d), jnp.bfloat16),
            jax.ShapeDtypeStruct((s_full, h, d), jnp.bfloat16),
        ),
        in_specs=[
            pl.BlockSpec(memory_space=pltpu.VMEM),
            pl.BlockSpec(memory_space=pltpu.VMEM),
        ],
        out_specs=(
            pl.BlockSpec(memory_space=pltpu.VMEM),
            pl.BlockSpec(memory_space=pltpu.VMEM),
        ),
        scratch_shapes=[
            pltpu.SemaphoreType.DMA((2,)),
            pltpu.SemaphoreType.DMA((2,)),
        ],
        compiler_params=pltpu.CompilerParams(collective_id=0),
    )(k, v)

    out = pl.pallas_call(
        _attn_body,
        grid=(h,),
        out_shape=jax.ShapeDtypeStruct((s_half, h, d), jnp.float32),
        in_specs=[
            pl.BlockSpec((s_half, 1, d), lambda i: (0, i, 0)),
            pl.BlockSpec((s_full, 1, d), lambda i: (0, i, 0)),
            pl.BlockSpec((s_full, 1, d), lambda i: (0, i, 0)),
        ],
        out_specs=pl.BlockSpec((s_half, 1, d), lambda i: (0, i, 0)),
    )(q, kf, vf)

    return out[None]


# baseline (device time: 232490 ns/iter reference)
import jax
import jax.numpy as jnp
from jax import lax
from jax.experimental import pallas as pl
from jax.experimental.pallas import tpu as pltpu


def _comm_body(q_ref, k_ref, v_ref, qh_ref, kf_ref, vf_ref,
               send_sems, recv_sems):
    s_half, n_heads, d = q_ref.shape
    my_x = lax.axis_index("x")
    my_y = lax.axis_index("y")
    my_z = lax.axis_index("z")
    other = (1 - my_x, my_y, my_z)

    scale = d ** -0.5
    for h in range(n_heads):
        qh_ref[h] = (q_ref[:, h, :] * scale).astype(jnp.bfloat16)
        kf_ref[my_x, h] = k_ref[:, h, :].astype(jnp.bfloat16)
        vf_ref[my_x, h] = v_ref[:, h, :].astype(jnp.bfloat16)

    barrier_sem = pltpu.get_barrier_semaphore()
    pl.semaphore_signal(
        barrier_sem, inc=1, device_id=other,
        device_id_type=pl.DeviceIdType.MESH,
    )
    pl.semaphore_wait(barrier_sem, 1)

    rdma_k = pltpu.make_async_remote_copy(
        src_ref=kf_ref.at[my_x],
        dst_ref=kf_ref.at[my_x],
        send_sem=send_sems.at[0],
        recv_sem=recv_sems.at[0],
        device_id=other,
        device_id_type=pl.DeviceIdType.MESH,
    )
    rdma_v = pltpu.make_async_remote_copy(
        src_ref=vf_ref.at[my_x],
        dst_ref=vf_ref.at[my_x],
        send_sem=send_sems.at[1],
        recv_sem=recv_sems.at[1],
        device_id=other,
        device_id_type=pl.DeviceIdType.MESH,
    )
    rdma_k.start()
    rdma_v.start()
    rdma_k.wait()
    rdma_v.wait()


def _attn_body(qh_ref, kf_ref, vf_ref, oh_ref):
    q = qh_ref[0]
    k = jnp.concatenate([kf_ref[0, 0], kf_ref[1, 0]], axis=0)
    s = lax.dot_general(
        q, k, (((1,), (1,)), ((), ())), preferred_element_type=jnp.float32
    )
    m = jnp.max(s, axis=1, keepdims=True)
    p = jnp.exp(s - m)
    l = jnp.sum(p, axis=1, keepdims=True)
    v = jnp.concatenate([vf_ref[0, 0], vf_ref[1, 0]], axis=0)
    o = lax.dot_general(
        p.astype(jnp.bfloat16), v, (((1,), (0,)), ((), ())),
        preferred_element_type=jnp.float32,
    )
    oh_ref[0] = o / l


def _untranspose_body(oh_ref, o_ref):
    n_heads = oh_ref.shape[0]
    for h in range(n_heads):
        o_ref[:, h, :] = oh_ref[h]


def kernel(Q, K, V):
    q = Q[0]
    k = K[0]
    v = V[0]
    s_half, h, d = q.shape

    qh, kf, vf = pl.pallas_call(
        _comm_body,
        out_shape=(
            jax.ShapeDtypeStruct((h, s_half, d), jnp.bfloat16),
            jax.ShapeDtypeStruct((2, h, s_half, d), jnp.bfloat16),
            jax.ShapeDtypeStruct((2, h, s_half, d), jnp.bfloat16),
        ),
        in_specs=[pl.BlockSpec(memory_space=pltpu.VMEM)] * 3,
        out_specs=(pl.BlockSpec(memory_space=pltpu.VMEM),) * 3,
        scratch_shapes=[
            pltpu.SemaphoreType.DMA((2,)),
            pltpu.SemaphoreType.DMA((2,)),
        ],
        compiler_params=pltpu.CompilerParams(collective_id=0),
    )(q, k, v)

    oh = pl.pallas_call(
        _attn_body,
        grid=(h,),
        out_shape=jax.ShapeDtypeStruct((h, s_half, d), jnp.float32),
        in_specs=[
            pl.BlockSpec((1, s_half, d), lambda i: (i, 0, 0)),
            pl.BlockSpec((2, 1, s_half, d), lambda i: (0, i, 0, 0)),
            pl.BlockSpec((2, 1, s_half, d), lambda i: (0, i, 0, 0)),
        ],
        out_specs=pl.BlockSpec((1, s_half, d), lambda i: (i, 0, 0)),
    )(qh, kf, vf)

    out = pl.pallas_call(
        _untranspose_body,
        out_shape=jax.ShapeDtypeStruct((s_half, h, d), jnp.float32),
        in_specs=[pl.BlockSpec(memory_space=pltpu.VMEM)],
        out_specs=pl.BlockSpec(memory_space=pltpu.VMEM),
    )(oh)

    return out[None]


# device time: 129274 ns/iter; 1.7984x vs baseline; 1.7984x over previous
import jax
import jax.numpy as jnp
from jax import lax
from jax.experimental import pallas as pl
from jax.experimental.pallas import tpu as pltpu


def _body(q_ref, k_ref, v_ref, o_ref,
          ks_ref, vs_ref, kr_ref, vr_ref, oa_ref, la_ref,
          ksend, vsend, krecv, vrecv):
    phase = pl.program_id(0)
    h = pl.program_id(1)

    my_x = lax.axis_index("x")
    my_y = lax.axis_index("y")
    my_z = lax.axis_index("z")
    other = (1 - my_x, my_y, my_z)

    d = q_ref.shape[-1]
    scale = d ** -0.5

    def _band_rdma(src, dst, s_sem, r_sem):
        return pltpu.make_async_remote_copy(
            src_ref=src, dst_ref=dst, send_sem=s_sem, recv_sem=r_sem,
            device_id=other, device_id_type=pl.DeviceIdType.MESH,
        )

    @pl.when(phase == 0)
    def _phase0():
        @pl.when(h == 0)
        def _():
            barrier_sem = pltpu.get_barrier_semaphore()
            pl.semaphore_signal(
                barrier_sem, inc=1, device_id=other,
                device_id_type=pl.DeviceIdType.MESH,
            )
            pl.semaphore_wait(barrier_sem, 1)

        ks_ref[h] = k_ref[...].astype(jnp.bfloat16)
        _band_rdma(ks_ref.at[h], kr_ref.at[h],
                   ksend.at[h], krecv.at[h]).start()
        vs_ref[h] = v_ref[...].astype(jnp.bfloat16)
        _band_rdma(vs_ref.at[h], vr_ref.at[h],
                   vsend.at[h], vrecv.at[h]).start()

        qh = (q_ref[...] * scale).astype(jnp.bfloat16)
        s0 = lax.dot_general(
            qh, ks_ref[h], (((1,), (1,)), ((), ())),
            preferred_element_type=jnp.float32,
        )
        p0 = jnp.exp(s0)
        la_ref[h] = jnp.sum(p0, axis=1, keepdims=True)
        oa_ref[h] = lax.dot_general(
            p0.astype(jnp.bfloat16), vs_ref[h], (((1,), (0,)), ((), ())),
            preferred_element_type=jnp.float32,
        )

    @pl.when(phase == 1)
    def _phase1():
        _band_rdma(ks_ref.at[h], kr_ref.at[h],
                   ksend.at[h], krecv.at[h]).wait()
        _band_rdma(vs_ref.at[h], vr_ref.at[h],
                   vsend.at[h], vrecv.at[h]).wait()

        qh = (q_ref[...] * scale).astype(jnp.bfloat16)
        s1 = lax.dot_general(
            qh, kr_ref[h], (((1,), (1,)), ((), ())),
            preferred_element_type=jnp.float32,
        )
        p1 = jnp.exp(s1)
        l1 = jnp.sum(p1, axis=1, keepdims=True)
        o1 = lax.dot_general(
            p1.astype(jnp.bfloat16), vr_ref[h], (((1,), (0,)), ((), ())),
            preferred_element_type=jnp.float32,
        )
        o_ref[...] = (oa_ref[h] + o1) / (la_ref[h] + l1)


def kernel(Q, K, V):
    _, s_half, h, d = Q.shape
    hd = h * d
    q = Q.reshape(s_half, hd)
    k = K.reshape(s_half, hd)
    v = V.reshape(s_half, hd)

    out = pl.pallas_call(
        _body,
        grid=(2, h),
        out_shape=jax.ShapeDtypeStruct((s_half, hd), jnp.float32),
        in_specs=[
            pl.BlockSpec((s_half, d), lambda p, i: (0, i)),
            pl.BlockSpec((s_half, d), lambda p, i: (0, i)),
            pl.BlockSpec((s_half, d), lambda p, i: (0, i)),
        ],
        out_specs=pl.BlockSpec((s_half, d), lambda p, i: (0, i)),
        scratch_shapes=[
            pltpu.VMEM((h, s_half, d), jnp.bfloat16),
            pltpu.VMEM((h, s_half, d), jnp.bfloat16),
            pltpu.VMEM((h, s_half, d), jnp.bfloat16),
            pltpu.VMEM((h, s_half, d), jnp.bfloat16),
            pltpu.VMEM((h, s_half, d), jnp.float32),
            pltpu.VMEM((h, s_half, 1), jnp.float32),
            pltpu.SemaphoreType.DMA((h,)),
            pltpu.SemaphoreType.DMA((h,)),
            pltpu.SemaphoreType.DMA((h,)),
            pltpu.SemaphoreType.DMA((h,)),
        ],
        compiler_params=pltpu.CompilerParams(
            collective_id=0, vmem_limit_bytes=60 * 1024 * 1024,
        ),
    )(q, k, v)

    return out.reshape(1, s_half, h, d)


# device time: 104775 ns/iter; 2.2189x vs baseline; 1.2338x over previous
import jax
import jax.numpy as jnp
from jax import lax
from jax.experimental import pallas as pl
from jax.experimental.pallas import tpu as pltpu

N_HEADS = 16


def _body(q_hbm, k_hbm, v_hbm, o_hbm,
          qs, qb, kb, vb, ks, vs, kr, vr, oa, la, ob,
          load_sems, store_sems, ksend, vsend, krecv, vrecv):
    phase = pl.program_id(0)
    h = pl.program_id(1)
    slot = lax.rem(h, 2)

    my_x = lax.axis_index("x")
    my_y = lax.axis_index("y")
    my_z = lax.axis_index("z")
    other = (1 - my_x, my_y, my_z)

    d = q_hbm.shape[-1]
    scale = d ** -0.5

    def load(t, band, slot_):
        src = (q_hbm, k_hbm, v_hbm)[t]
        dst = (qb, kb, vb)[t]
        return pltpu.make_async_copy(
            src.at[0, :, band, :], dst.at[slot_], load_sems.at[slot_, t])

    def band_rdma(sref, rref, s_sem, r_sem):
        return pltpu.make_async_remote_copy(
            src_ref=sref.at[h], dst_ref=rref.at[h],
            send_sem=s_sem.at[h], recv_sem=r_sem.at[h],
            device_id=other, device_id_type=pl.DeviceIdType.MESH)

    def store(band, slot_):
        return pltpu.make_async_copy(
            ob.at[slot_], o_hbm.at[0, :, band, :], store_sems.at[slot_])

    @pl.when(phase == 0)
    def _phase0():
        @pl.when(h == 0)
        def _():
            for t in range(3):
                load(t, 0, 0).start()
                load(t, 1, 1).start()
            barrier_sem = pltpu.get_barrier_semaphore()
            pl.semaphore_signal(
                barrier_sem, inc=1, device_id=other,
                device_id_type=pl.DeviceIdType.MESH)
            pl.semaphore_wait(barrier_sem, 1)

        for t in range(3):
            load(t, h, slot).wait()

        ks[h] = kb[slot].astype(jnp.bfloat16)
        band_rdma(ks, kr, ksend, krecv).start()
        vs[h] = vb[slot].astype(jnp.bfloat16)
        band_rdma(vs, vr, vsend, vrecv).start()
        qs[h] = (qb[slot] * scale).astype(jnp.bfloat16)

        s0 = lax.dot_general(
            qs[h], ks[h], (((1,), (1,)), ((), ())),
            preferred_element_type=jnp.float32)
        p0 = jnp.exp(s0)
        la[h] = jnp.sum(p0, axis=1, keepdims=True)
        oa[h] = lax.dot_general(
            p0.astype(jnp.bfloat16), vs[h], (((1,), (0,)), ((), ())),
            preferred_element_type=jnp.float32)

        @pl.when(h + 2 < N_HEADS)
        def _():
            for t in range(3):
                load(t, h + 2, slot).start()

    @pl.when(phase == 1)
    def _phase1():
        band_rdma(ks, kr, ksend, krecv).wait()
        band_rdma(vs, vr, vsend, vrecv).wait()

        s1 = lax.dot_general(
            qs[h], kr[h], (((1,), (1,)), ((), ())),
            preferred_element_type=jnp.float32)
        p1 = jnp.exp(s1)
        l1 = jnp.sum(p1, axis=1, keepdims=True)
        o1 = lax.dot_general(
            p1.astype(jnp.bfloat16), vr[h], (((1,), (0,)), ((), ())),
            preferred_element_type=jnp.float32)

        @pl.when(h >= 2)
        def _():
            store(h - 2, slot).wait()
        ob[slot] = (oa[h] + o1) / (la[h] + l1)
        store(h, slot).start()

        @pl.when(h == N_HEADS - 1)
        def _():
            store(h - 1, 1 - slot).wait()
            store(h, slot).wait()


def kernel(Q, K, V):
    _, s_half, h, d = Q.shape

    out = pl.pallas_call(
        _body,
        grid=(2, h),
        out_shape=jax.ShapeDtypeStruct((1, s_half, h, d), jnp.float32),
        in_specs=[pl.BlockSpec(memory_space=pl.ANY)] * 3,
        out_specs=pl.BlockSpec(memory_space=pl.ANY),
        scratch_shapes=[
            pltpu.VMEM((h, s_half, d), jnp.bfloat16),
            pltpu.VMEM((2, s_half, d), jnp.float32),
            pltpu.VMEM((2, s_half, d), jnp.float32),
            pltpu.VMEM((2, s_half, d), jnp.float32),
            pltpu.VMEM((h, s_half, d), jnp.bfloat16),
            pltpu.VMEM((h, s_half, d), jnp.bfloat16),
            pltpu.VMEM((h, s_half, d), jnp.bfloat16),
            pltpu.VMEM((h, s_half, d), jnp.bfloat16),
            pltpu.VMEM((h, s_half, d), jnp.float32),
            pltpu.VMEM((h, s_half, 1), jnp.float32),
            pltpu.VMEM((2, s_half, d), jnp.float32),
            pltpu.SemaphoreType.DMA((2, 3)),
            pltpu.SemaphoreType.DMA((2,)),
            pltpu.SemaphoreType.DMA((h,)),
            pltpu.SemaphoreType.DMA((h,)),
            pltpu.SemaphoreType.DMA((h,)),
            pltpu.SemaphoreType.DMA((h,)),
        ],
        compiler_params=pltpu.CompilerParams(
            collective_id=0, vmem_limit_bytes=62 * 1024 * 1024),
    )(Q, K, V)

    return out
